# baseline (speedup 1.0000x reference)
import numpy as np

import jax
import jax.numpy as jnp
from jax.experimental import pallas as pl
from jax.experimental.pallas import tpu as pltpu

B = 4096
S = 12
D = 1024
K = 3

BB = 64


def _threefry2x32(k1, k2, x1, x2):
    rot = ((13, 15, 26, 6), (17, 29, 16, 24))
    ks = (k1, k2, np.uint32(k1 ^ k2 ^ np.uint32(0x1BD11BDA)))
    x = [x1 + ks[0], x2 + ks[1]]
    for i in range(5):
        for r in rot[i % 2]:
            x[0] = x[0] + x[1]
            x[1] = (x[1] << np.uint32(r)) | (x[1] >> np.uint32(32 - r))
            x[1] = x[0] ^ x[1]
        x[0] = x[0] + ks[(i + 1) % 3]
        x[1] = x[1] + ks[(i + 2) % 3] + np.uint32(i + 1)
    return x[0], x[1]


def _uniform_np(seed: int, shape) -> np.ndarray:
    k1, k2 = np.uint32(seed >> 32), np.uint32(seed & 0xFFFFFFFF)
    n = int(np.prod(shape))
    flat = np.arange(n, dtype=np.uint64)
    c1 = (flat >> np.uint64(32)).astype(np.uint32)
    c2 = (flat & np.uint64(0xFFFFFFFF)).astype(np.uint32)
    b1, b2 = _threefry2x32(k1, k2, c1, c2)
    bits = b1 ^ b2
    fb = (bits >> np.uint32(9)) | np.uint32(0x3F800000)
    return (fb.view(np.float32) - np.float32(1.0)).reshape(shape)


def _selection() -> np.ndarray:
    rand = _uniform_np(1, (B, S))
    return np.argsort(rand, axis=-1, kind="stable")[:, :K].astype(np.int32)


_SEL = _selection()
_SELR = np.repeat(_SEL[:, :, None], 128, axis=2)

DT = 128


def _body(in_ref, selr_ref, out_ref):
    mks = [selr_ref[:, k, :] for k in range(K)]
    for dt in range(D // DT):
        ds_ = slice(dt * DT, (dt + 1) * DT)
        x = in_ref[:, 0, ds_]
        accs = [x, x, x]
        for s in range(1, S):
            x = in_ref[:, s, ds_]
            for k in range(K):
                accs[k] = jnp.where(mks[k] == s, x, accs[k])
        for k in range(K):
            out_ref[:, k, ds_] = accs[k]


def kernel(image_latent):
    return pl.pallas_call(
        _body,
        grid=(B // BB,),
        in_specs=[
            pl.BlockSpec((BB, S, D), lambda g: (g, 0, 0)),
            pl.BlockSpec((BB, K, DT), lambda g: (g, 0, 0)),
        ],
        out_specs=pl.BlockSpec((BB, K, D), lambda g: (g, 0, 0)),
        out_shape=jax.ShapeDtypeStruct((B, K, D), jnp.float32),
        compiler_params=pltpu.CompilerParams(
            dimension_semantics=("arbitrary",),
        ),
    )(image_latent, jnp.asarray(_SELR))


# speedup vs baseline: 1.7049x; 1.7049x over previous
import numpy as np

import jax
import jax.numpy as jnp
from jax.experimental import pallas as pl
from jax.experimental.pallas import tpu as pltpu

B = 4096
S = 12
D = 1024
K = 3

BB = 64


def _threefry2x32(k1, k2, x1, x2):
    rot = ((13, 15, 26, 6), (17, 29, 16, 24))
    ks = (k1, k2, np.uint32(k1 ^ k2 ^ np.uint32(0x1BD11BDA)))
    x = [x1 + ks[0], x2 + ks[1]]
    for i in range(5):
        for r in rot[i % 2]:
            x[0] = x[0] + x[1]
            x[1] = (x[1] << np.uint32(r)) | (x[1] >> np.uint32(32 - r))
            x[1] = x[0] ^ x[1]
        x[0] = x[0] + ks[(i + 1) % 3]
        x[1] = x[1] + ks[(i + 2) % 3] + np.uint32(i + 1)
    return x[0], x[1]


def _uniform_np(seed: int, shape) -> np.ndarray:
    k1, k2 = np.uint32(seed >> 32), np.uint32(seed & 0xFFFFFFFF)
    n = int(np.prod(shape))
    flat = np.arange(n, dtype=np.uint64)
    c1 = (flat >> np.uint64(32)).astype(np.uint32)
    c2 = (flat & np.uint64(0xFFFFFFFF)).astype(np.uint32)
    b1, b2 = _threefry2x32(k1, k2, c1, c2)
    bits = b1 ^ b2
    fb = (bits >> np.uint32(9)) | np.uint32(0x3F800000)
    return (fb.view(np.float32) - np.float32(1.0)).reshape(shape)


def _selection() -> np.ndarray:
    rand = _uniform_np(1, (B, S))
    return np.argsort(rand, axis=-1, kind="stable")[:, :K].astype(np.int32)


_SEL = _selection()
_OH = np.zeros((B, K, S), dtype=np.float32)
_OH[np.arange(B)[:, None], np.arange(K)[None, :], _SEL] = 1.0


def _body(in_ref, oh_ref, out_ref):
    for bl in range(BB):
        out_ref[bl] = jnp.dot(
            oh_ref[bl], in_ref[bl], preferred_element_type=jnp.float32
        )


def kernel(image_latent):
    return pl.pallas_call(
        _body,
        grid=(B // BB,),
        in_specs=[
            pl.BlockSpec((BB, S, D), lambda g: (g, 0, 0)),
            pl.BlockSpec((BB, K, S), lambda g: (g, 0, 0)),
        ],
        out_specs=pl.BlockSpec((BB, K, D), lambda g: (g, 0, 0)),
        out_shape=jax.ShapeDtypeStruct((B, K, D), jnp.float32),
        compiler_params=pltpu.CompilerParams(
            dimension_semantics=("arbitrary",),
        ),
    )(image_latent, jnp.asarray(_OH))
